# Initial kernel scaffold; baseline (speedup 1.0000x reference)
import jax
import jax.numpy as jnp
from jax.experimental import pallas as pl


def kernel(x, pos, W1a, b1a, W1b, b1b, W2a, b2a, W2b, b2b, W3a, b3a, W3b, b3b):
    raise NotImplementedError("write your pallas kernel here")


# speedup vs baseline: 8.5255x; 8.5255x over previous
import functools

import numpy as np
import jax
import jax.numpy as jnp
from jax.experimental import pallas as pl
from jax.experimental.pallas import tpu as pltpu

_B = 4
_N1 = 1024
_M1 = 256
_M2 = 64
_R2_1 = 0.4 * 0.4
_R2_2 = 0.8 * 0.8


def _fps_kernel(px_ref, py_ref, q1x_ref, q1y_ref, q2x_ref, q2y_ref):
    def run(px, py, m, qx_ref, qy_ref):
        b, n = px.shape
        iota = jax.lax.broadcasted_iota(jnp.int32, (1, n), 1)
        miota = jax.lax.broadcasted_iota(jnp.int32, (1, m), 1)

        def body(i, carry):
            dist, last, qx, qy = carry
            sel = (iota == last).astype(jnp.float32)
            lx = jnp.sum(px * sel, axis=1, keepdims=True)
            ly = jnp.sum(py * sel, axis=1, keepdims=True)
            at = miota == (i - 1)
            qx = jnp.where(at, lx, qx)
            qy = jnp.where(at, ly, qy)
            d = (px - lx) ** 2 + (py - ly) ** 2
            dist = jnp.minimum(dist, d)
            mx = jnp.max(dist, axis=1, keepdims=True)
            nxt = jnp.min(jnp.where(dist == mx, iota, n), axis=1, keepdims=True)
            return dist, nxt, qx, qy

        dist0 = jnp.full((b, n), jnp.inf, dtype=jnp.float32)
        last0 = jnp.zeros((b, 1), dtype=jnp.int32)
        qx0 = jnp.zeros((b, m), dtype=jnp.float32)
        _, _, qx, qy = jax.lax.fori_loop(1, m + 1, body,
                                         (dist0, last0, qx0, qx0))
        qx_ref[...] = qx
        qy_ref[...] = qy

    run(px_ref[...], py_ref[...], _M1, q1x_ref, q1y_ref)
    run(q1x_ref[...], q1y_ref[...], _M2, q2x_ref, q2y_ref)


def _fps(px, py):
    f32 = jnp.float32
    return pl.pallas_call(
        _fps_kernel,
        out_shape=[
            jax.ShapeDtypeStruct((_B, _M1), f32),
            jax.ShapeDtypeStruct((_B, _M1), f32),
            jax.ShapeDtypeStruct((_B, _M2), f32),
            jax.ShapeDtypeStruct((_B, _M2), f32),
        ],
    )(px, py)


def _lin_kernel(x_ref, w_ref, b_ref, o_ref):
    x = x_ref[...].reshape(x_ref.shape[1:])
    o = jnp.dot(x, w_ref[...], preferred_element_type=jnp.float32) + b_ref[...]
    o_ref[...] = o.reshape(o_ref.shape)


def _lin(x, w, b):
    bsz, n, c = x.shape
    h = w.shape[1]
    return pl.pallas_call(
        _lin_kernel,
        grid=(bsz,),
        in_specs=[
            pl.BlockSpec((1, n, c), lambda i: (i, 0, 0)),
            pl.BlockSpec((c, h), lambda i: (0, 0)),
            pl.BlockSpec((1, h), lambda i: (0, 0)),
        ],
        out_specs=pl.BlockSpec((1, n, h), lambda i: (i, 0, 0)),
        out_shape=jax.ShapeDtypeStruct((bsz, n, h), jnp.float32),
    )(x, w, b)


def _sa_kernel(u_ref, sx_ref, sy_ref, qx_ref, qy_ref, wpx_ref, wpy_ref,
               wb_ref, bb_ref, o_ref, acc_ref, *, r2, ns):
    s = pl.program_id(2)

    @pl.when(s == 0)
    def _():
        acc_ref[...] = jnp.full(acc_ref.shape, -jnp.inf, dtype=jnp.float32)

    u = u_ref[...].reshape(u_ref.shape[1:])
    sx = sx_ref[...].reshape(sx_ref.shape[2:])
    sy = sy_ref[...].reshape(sy_ref.shape[2:])
    qx = qx_ref[...].reshape(qx_ref.shape[1:])
    qy = qy_ref[...].reshape(qy_ref.shape[1:])
    relx = sx - qx
    rely = sy - qy
    tq, ssz = relx.shape
    h = u.shape[1]
    wpx = wpx_ref[...].reshape(1, 1, h)
    wpy = wpy_ref[...].reshape(1, 1, h)
    pre = (u[None, :, :]
           + relx[:, :, None] * wpx
           + rely[:, :, None] * wpy)
    t = jnp.tanh(pre).reshape(tq * ssz, h)
    ho = wb_ref.shape[1]
    hh = jnp.dot(t, wb_ref[...], preferred_element_type=jnp.float32)
    hh = (hh + bb_ref[...]).reshape(tq, ssz, ho)
    d2 = relx * relx + rely * rely
    pen = jnp.where(d2 <= r2, 0.0, -jnp.inf).astype(jnp.float32)
    hm = hh + pen[:, :, None]
    acc_ref[...] = jnp.maximum(acc_ref[...], jnp.max(hm, axis=1))

    @pl.when(s == ns - 1)
    def _():
        o_ref[...] = acc_ref[...].reshape(o_ref.shape)


def _sa(u, sx, sy, qx3, qy3, wpx, wpy, wb, bb, r2, tq, schunk):
    bsz, n, h = u.shape
    m = qx3.shape[1]
    ho = wb.shape[1]
    nq = m // tq
    ns = n // schunk
    sx4 = sx.reshape(bsz, ns, 1, schunk)
    sy4 = sy.reshape(bsz, ns, 1, schunk)
    kern = functools.partial(_sa_kernel, r2=r2, ns=ns)
    return pl.pallas_call(
        kern,
        grid=(bsz, nq, ns),
        in_specs=[
            pl.BlockSpec((1, schunk, h), lambda b, q, s: (b, s, 0)),
            pl.BlockSpec((1, 1, 1, schunk), lambda b, q, s: (b, s, 0, 0)),
            pl.BlockSpec((1, 1, 1, schunk), lambda b, q, s: (b, s, 0, 0)),
            pl.BlockSpec((1, tq, 1), lambda b, q, s: (b, q, 0)),
            pl.BlockSpec((1, tq, 1), lambda b, q, s: (b, q, 0)),
            pl.BlockSpec((1, h), lambda b, q, s: (0, 0)),
            pl.BlockSpec((1, h), lambda b, q, s: (0, 0)),
            pl.BlockSpec((h, ho), lambda b, q, s: (0, 0)),
            pl.BlockSpec((1, ho), lambda b, q, s: (0, 0)),
        ],
        out_specs=pl.BlockSpec((1, tq, ho), lambda b, q, s: (b, q, 0)),
        out_shape=jax.ShapeDtypeStruct((bsz, m, ho), jnp.float32),
        scratch_shapes=[pltpu.VMEM((tq, ho), jnp.float32)],
        compiler_params=pltpu.CompilerParams(
            dimension_semantics=("parallel", "parallel", "arbitrary"),
        ),
    )(u, sx4, sy4, qx3, qy3, wpx, wpy, wb, bb)


def _global_kernel(x2_ref, qx_ref, qy_ref, w3x_ref, w3px_ref, w3py_ref,
                   b3a_ref, w3b_ref, b3b_ref, o_ref):
    x2 = x2_ref[...].reshape(x2_ref.shape[1:])
    qx = qx_ref[...].reshape(qx_ref.shape[1:])
    qy = qy_ref[...].reshape(qy_ref.shape[1:])
    pre = jnp.dot(x2, w3x_ref[...], preferred_element_type=jnp.float32)
    pre = pre + qx * w3px_ref[...] + qy * w3py_ref[...] + b3a_ref[...]
    hh = jnp.dot(jnp.tanh(pre), w3b_ref[...],
                 preferred_element_type=jnp.float32) + b3b_ref[...]
    o_ref[...] = jnp.max(hh, axis=0, keepdims=True)[None]


def _global(x2, qx3, qy3, w3x, w3px, w3py, b3a, w3b, b3b):
    bsz, m, c = x2.shape
    h1 = w3x.shape[1]
    h2 = w3b.shape[1]
    return pl.pallas_call(
        _global_kernel,
        grid=(bsz,),
        in_specs=[
            pl.BlockSpec((1, m, c), lambda i: (i, 0, 0)),
            pl.BlockSpec((1, m, 1), lambda i: (i, 0, 0)),
            pl.BlockSpec((1, m, 1), lambda i: (i, 0, 0)),
            pl.BlockSpec((c, h1), lambda i: (0, 0)),
            pl.BlockSpec((1, h1), lambda i: (0, 0)),
            pl.BlockSpec((1, h1), lambda i: (0, 0)),
            pl.BlockSpec((1, h1), lambda i: (0, 0)),
            pl.BlockSpec((h1, h2), lambda i: (0, 0)),
            pl.BlockSpec((1, h2), lambda i: (0, 0)),
        ],
        out_specs=pl.BlockSpec((1, 1, h2), lambda i: (i, 0, 0)),
        out_shape=jax.ShapeDtypeStruct((bsz, 1, h2), jnp.float32),
    )(x2, qx3, qy3, w3x, w3px, w3py, b3a, w3b, b3b)


def kernel(x, pos, W1a, b1a, W1b, b1b, W2a, b2a, W2b, b2b, W3a, b3a, W3b, b3b):
    px = pos[:, :, 0]
    py = pos[:, :, 1]
    q1x, q1y, q2x, q2y = _fps(px, py)

    u1 = _lin(x, W1a[:64], b1a[None])
    x1 = _sa(u1, px, py, q1x[:, :, None], q1y[:, :, None],
             W1a[64:65], W1a[65:66], W1b, b1b[None],
             r2=_R2_1, tq=64, schunk=128)

    u2 = _lin(x1, W2a[:128], b2a[None])
    x2 = _sa(u2, q1x, q1y, q2x[:, :, None], q2y[:, :, None],
             W2a[128:129], W2a[129:130], W2b, b2b[None],
             r2=_R2_2, tq=64, schunk=64)

    out = _global(x2, q2x[:, :, None], q2y[:, :, None],
                  W3a[:256], W3a[256:257], W3a[257:258], b3a[None],
                  W3b, b3b[None])
    return out.reshape(_B, -1)
